# baseline (speedup 1.0000x reference)
import functools

import jax
import jax.numpy as jnp
from jax.experimental import pallas as pl
from jax.experimental.pallas import tpu as pltpu


def _gather_concat_kernel(ids_ref, wtbl_ref, ptbl_ref, out_ref, *, T, S, U):
    Dw = wtbl_ref.shape[2]
    tile = pl.program_id(0)
    base = tile * T

    def chunk(c, carry):
        i0 = c * U
        r0 = c * (U // 8)
        rows = [wtbl_ref[ids_ref[base + i0 + u], 0] for u in range(U)]
        for u in range(0, U, 2):
            pair = jnp.stack([rows[u], rows[u + 1]], axis=0)
            out_ref[r0 + u // 8, (u % 8):(u % 8) + 2, 0:Dw] = pair
        return carry

    jax.lax.fori_loop(0, T // U, chunk, 0)

    if T % S == 0:
        for k in range(T // S):
            out_ref[k * (S // 8):(k + 1) * (S // 8), :, Dw:] = \
                ptbl_ref[0:S // 8, :, :]
    else:
        pos0 = (base % S) // 8
        out_ref[:, :, Dw:] = ptbl_ref[pl.ds(pos0, T // 8), :, :]


def kernel(input_ids, word_table, pos_table):
    B, S = input_ids.shape
    V, Dw = word_table.shape
    P, Dp = pos_table.shape
    n = B * S
    Dout = Dw + Dp

    T = S
    while T > 2048 or T % 8 != 0:
        T //= 2
    while (T < 2048 and n % (2 * T) == 0
           and ((2 * T) % S == 0 or S % (2 * T) == 0)):
        T *= 2
    n_tiles = pl.cdiv(n, T)

    ids = input_ids.reshape(n).astype(jnp.int32)
    wtbl3 = word_table.reshape(V, 1, Dw)
    ptbl3 = pos_table.reshape(P // 8, 8, Dp)

    U = 512
    while U > T:
        U //= 2

    itemsize = word_table.dtype.itemsize
    table_bytes = (word_table.size + pos_table.size) * itemsize
    out_tile_bytes = T * Dout * itemsize
    vmem_limit = int(min(table_bytes + 8 * out_tile_bytes + (4 << 20),
                         56 << 20))

    grid_spec = pltpu.PrefetchScalarGridSpec(
        num_scalar_prefetch=1,
        grid=(n_tiles,),
        in_specs=[
            pl.BlockSpec((V, 1, Dw), lambda i, ids: (0, 0, 0)),
            pl.BlockSpec((P // 8, 8, Dp), lambda i, ids: (0, 0, 0)),
        ],
        out_specs=pl.BlockSpec((T // 8, 8, Dout), lambda i, ids: (i, 0, 0)),
    )

    out_flat = pl.pallas_call(
        functools.partial(_gather_concat_kernel, T=T, S=S, U=U),
        out_shape=jax.ShapeDtypeStruct((n // 8, 8, Dout), word_table.dtype),
        grid_spec=grid_spec,
        compiler_params=pltpu.CompilerParams(
            dimension_semantics=("parallel",),
            vmem_limit_bytes=vmem_limit),
    )(ids, wtbl3, ptbl3)

    return out_flat.reshape(B, S, Dout)


# speedup vs baseline: 1.0040x; 1.0040x over previous
import functools

import jax
import jax.numpy as jnp
from jax.experimental import pallas as pl
from jax.experimental.pallas import tpu as pltpu


def _gather_concat_kernel(ids_ref, wtbl_ref, ptbl_ref, out_ref, *, T, S, U):
    Dw = wtbl_ref.shape[2]
    tile = pl.program_id(0)
    base = tile * T

    def chunk(c, carry):
        i0 = c * U
        r0 = c * (U // 8)
        rows = [wtbl_ref[ids_ref[base + i0 + u], 0] for u in range(U)]
        for u in range(U):
            out_ref[r0 + u // 8, u % 8, 0:Dw] = rows[u]
        return carry

    jax.lax.fori_loop(0, T // U, chunk, 0)

    if T % S == 0:
        for k in range(T // S):
            out_ref[k * (S // 8):(k + 1) * (S // 8), :, Dw:] = \
                ptbl_ref[0:S // 8, :, :]
    else:
        pos0 = (base % S) // 8
        out_ref[:, :, Dw:] = ptbl_ref[pl.ds(pos0, T // 8), :, :]


def kernel(input_ids, word_table, pos_table):
    B, S = input_ids.shape
    V, Dw = word_table.shape
    P, Dp = pos_table.shape
    n = B * S
    Dout = Dw + Dp

    T = S
    while T > 2048 or T % 8 != 0:
        T //= 2
    while (T < 2048 and n % (2 * T) == 0
           and ((2 * T) % S == 0 or S % (2 * T) == 0)):
        T *= 2
    n_tiles = pl.cdiv(n, T)

    ids = input_ids.reshape(n).astype(jnp.int32)
    wtbl3 = word_table.reshape(V, 1, Dw)
    ptbl3 = pos_table.reshape(P // 8, 8, Dp)

    U = 512
    while U > T:
        U //= 2

    itemsize = word_table.dtype.itemsize
    table_bytes = (word_table.size + pos_table.size) * itemsize
    out_tile_bytes = T * Dout * itemsize
    vmem_limit = int(min(table_bytes + 8 * out_tile_bytes + (4 << 20),
                         56 << 20))

    grid_spec = pltpu.PrefetchScalarGridSpec(
        num_scalar_prefetch=1,
        grid=(n_tiles,),
        in_specs=[
            pl.BlockSpec((V, 1, Dw), lambda i, ids: (0, 0, 0)),
            pl.BlockSpec((P // 8, 8, Dp), lambda i, ids: (0, 0, 0)),
        ],
        out_specs=pl.BlockSpec((T // 8, 8, Dout), lambda i, ids: (i, 0, 0)),
    )

    out_flat = pl.pallas_call(
        functools.partial(_gather_concat_kernel, T=T, S=S, U=U),
        out_shape=jax.ShapeDtypeStruct((n // 8, 8, Dout), word_table.dtype),
        grid_spec=grid_spec,
        compiler_params=pltpu.CompilerParams(
            dimension_semantics=("parallel",),
            vmem_limit_bytes=vmem_limit),
    )(ids, wtbl3, ptbl3)

    return out_flat.reshape(B, S, Dout)
